# Initial kernel scaffold; baseline (speedup 1.0000x reference)
import functools

import jax
import jax.numpy as jnp
from jax import lax
from jax.experimental import pallas as pl
from jax.experimental.pallas import tpu as pltpu
from jax.experimental.pallas import tpu_sc as plsc

_N = 10000
_R = 16
_D = 128
_E = 320000
_W = 144
_NC = 2
_NS = 16
_EPT = _E // _NS
_RPT = _N // _NS
_ZCH = 125
_CH = 128


def _sc_segment_sums(xa, edge_index, edge_type):
    mesh = plsc.VectorSubcoreMesh(
        core_axis_name="c", subcore_axis_name="s",
        num_cores=_NC, num_subcores=_NS)
    zrows = jnp.zeros((_ZCH, _W), jnp.float32)

    @functools.partial(
        pl.kernel,
        out_type=jax.ShapeDtypeStruct((_R, _N, _W), jnp.float32),
        mesh=mesh,
        scratch_types=[
            pltpu.VMEM((_EPT + 16,), jnp.int32),
            pltpu.VMEM((_EPT + 16,), jnp.int32),
            pltpu.VMEM((_EPT,), jnp.int32),
            pltpu.VMEM((_EPT + _CH,), jnp.int32),
            pltpu.VMEM((1, _CH), jnp.int32),
            pltpu.VMEM((1, _CH), jnp.int32),
            pltpu.VMEM((_CH, _W), jnp.float32),
            pltpu.VMEM((_ZCH, _W), jnp.float32),
            pltpu.VMEM_SHARED((_N, _W), jnp.float32),
            pltpu.SemaphoreType.DMA,
        ],
    )
    def k(xa_hbm, ei_hbm, et_hbm, zr_hbm, out_hbm,
          src_res, dst_res, et_res, comp_pos, src_idx, dst_idx, rows, zbuf,
          acc, sem):
        cid = lax.axis_index("c")
        tid = lax.axis_index("s")
        e0 = tid * _EPT
        pltpu.sync_copy(ei_hbm.at[0, pl.ds(e0, _EPT)], src_res.at[pl.ds(0, _EPT)])
        pltpu.sync_copy(ei_hbm.at[1, pl.ds(e0, _EPT)], dst_res.at[pl.ds(0, _EPT)])
        pltpu.sync_copy(et_hbm.at[pl.ds(e0, _EPT)], et_res)
        pltpu.sync_copy(zr_hbm, zbuf)
        src_res[pl.ds(_EPT, 16)] = jnp.full((16,), _N, jnp.int32)
        dst_res[pl.ds(_EPT, 16)] = jnp.zeros((16,), jnp.int32)
        r0 = tid * _RPT
        for z in range(_RPT // _ZCH):
            pltpu.sync_copy(zbuf, acc.at[pl.ds(r0 + z * _ZCH, _ZCH)])
        plsc.subcore_barrier()

        iota16 = lax.iota(jnp.int32, 16)
        dummy16 = jnp.full((16,), _EPT, jnp.int32)

        for p in range(_R // _NC):
            rel = _NC * p + cid

            def scan_body(i, kn, rel=rel):
                et16 = et_res[pl.ds(i * 16, 16)]
                m = et16 == rel
                pos16 = iota16 + i * 16
                plsc.store_compressed(comp_pos.at[pl.ds(kn, 16)], pos16, mask=m)
                return kn + jnp.sum(m.astype(jnp.int32))

            kn = lax.fori_loop(0, _EPT // 16, scan_body, jnp.int32(0))
            for v in range(_CH // 16):
                comp_pos[pl.ds(kn + v * 16, 16)] = dummy16
            nch = (kn + (_CH - 1)) // _CH

            def chunk_body(j, _):
                for v in range(_CH // 16):
                    idx16 = comp_pos[pl.ds(j * _CH + v * 16, 16)]
                    s16 = plsc.load_gather(src_res, [idx16])
                    d16 = plsc.load_gather(dst_res, [idx16])
                    src_idx.at[0][pl.ds(v * 16, 16)] = s16
                    dst_idx.at[0][pl.ds(v * 16, 16)] = d16
                pltpu.async_copy(xa_hbm.at[src_idx.at[0]], rows, sem).wait()
                pltpu.sync_copy(rows, acc.at[dst_idx.at[0]], add=True)
                return 0

            lax.fori_loop(0, nch, chunk_body, 0)
            plsc.subcore_barrier()
            for z in range(_RPT // _ZCH):
                sl = pl.ds(r0 + z * _ZCH, _ZCH)
                pltpu.sync_copy(acc.at[sl], out_hbm.at[rel].at[sl])
                pltpu.sync_copy(zbuf, acc.at[sl])
            plsc.subcore_barrier()

    return k(xa, edge_index, edge_type, zrows)


def _tc_body(sums_ref, x_ref, wd_ref, root_ref, b_ref, g_ref, bb_ref, out_ref,
             *, residual):
    xb = x_ref[...]
    acc = jnp.dot(xb, root_ref[...], preferred_element_type=jnp.float32)
    acc = acc + b_ref[...]
    for r in range(_R):
        sr = sums_ref[r]
        cnt = jnp.maximum(sr[:, 128:129], 1.0)
        mean = sr[:, :128] / cnt
        acc = acc + jnp.dot(mean, wd_ref[r], preferred_element_type=jnp.float32)
    if residual:
        acc = acc + xb
    mu = jnp.mean(acc, axis=-1, keepdims=True)
    var = jnp.mean((acc - mu) ** 2, axis=-1, keepdims=True)
    y = (acc - mu) * lax.rsqrt(var + 1e-5) * g_ref[...] + bb_ref[...]
    out_ref[...] = jnp.maximum(y, 0.0)


def _tc_layer(sums, x, wd, root, bias, g, bb, *, residual):
    nb = 1000
    grid = (_N // nb,)
    return pl.pallas_call(
        functools.partial(_tc_body, residual=residual),
        grid=grid,
        in_specs=[
            pl.BlockSpec((_R, nb, _W), lambda i: (0, i, 0)),
            pl.BlockSpec((nb, _D), lambda i: (i, 0)),
            pl.BlockSpec((_R, _D, _D), lambda i: (0, 0, 0)),
            pl.BlockSpec((_D, _D), lambda i: (0, 0)),
            pl.BlockSpec((1, _D), lambda i: (0, 0)),
            pl.BlockSpec((1, _D), lambda i: (0, 0)),
            pl.BlockSpec((1, _D), lambda i: (0, 0)),
        ],
        out_specs=pl.BlockSpec((nb, _D), lambda i: (i, 0)),
        out_shape=jax.ShapeDtypeStruct((_N, _D), jnp.float32),
    )(sums, x, wd, root, bias, g, bb)


def _expand_blockdiag(w):
    return jax.vmap(lambda wr: jax.scipy.linalg.block_diag(*[wr[b] for b in range(4)]))(w)


def _augment(x):
    xa = jnp.zeros((_N + 1, _W), jnp.float32)
    xa = xa.at[:_N, :_D].set(x)
    xa = xa.at[:_N, _D].set(1.0)
    return xa


def kernel(edge_index, edge_type, entity_emb, w0, root0, b0, ln_g0, ln_b0,
           w1, root1, b1, ln_g1, ln_b1):
    x = entity_emb
    wd0 = _expand_blockdiag(w0)
    wd1 = _expand_blockdiag(w1)
    sums0 = _sc_segment_sums(_augment(x), edge_index, edge_type)
    x1 = _tc_layer(sums0, x, wd0, root0, b0.reshape(1, -1),
                   ln_g0.reshape(1, -1), ln_b0.reshape(1, -1), residual=False)
    sums1 = _sc_segment_sums(_augment(x1), edge_index, edge_type)
    x2 = _tc_layer(sums1, x1, wd1, root1, b1.reshape(1, -1),
                   ln_g1.reshape(1, -1), ln_b1.reshape(1, -1), residual=True)
    return x2


# speedup vs baseline: 2.0983x; 2.0983x over previous
import functools

import jax
import jax.numpy as jnp
from jax import lax
from jax.experimental import pallas as pl
from jax.experimental.pallas import tpu as pltpu
from jax.experimental.pallas import tpu_sc as plsc

_N = 10000
_R = 16
_D = 128
_E = 320000
_W = 144
_NC = 2
_NS = 16
_EPT = _E // _NS
_NA = 10240
_RPT = _NA // _NS
_OCH = 128
_ZCH = 32
_CH = 128
_SL = 2000
_NSTRIP = _EPT // _SL
_CCAP = 2192
_CTRASH = 2176


def _sc_segment_sums(xa, src, dst, edge_type):
    mesh = plsc.VectorSubcoreMesh(
        core_axis_name="c", subcore_axis_name="s",
        num_cores=_NC, num_subcores=_NS)
    zrows = jnp.zeros((_ZCH, _W), jnp.float32)

    @functools.partial(
        pl.kernel,
        out_type=jax.ShapeDtypeStruct((_R, _N, _W), jnp.float32),
        mesh=mesh,
        scratch_types=[
            pltpu.VMEM((_SL,), jnp.int32),
            pltpu.VMEM((_SL,), jnp.int32),
            pltpu.VMEM((_SL,), jnp.int32),
            pltpu.VMEM((_CCAP,), jnp.int32),
            pltpu.VMEM((_CCAP,), jnp.int32),
            pltpu.VMEM((1, _CH), jnp.int32),
            pltpu.VMEM((1, _CH), jnp.int32),
            pltpu.VMEM((_CH, _W), jnp.float32),
            pltpu.VMEM((_ZCH, _W), jnp.float32),
            pltpu.VMEM_SHARED((_NA, _W), jnp.float32),
            pltpu.SemaphoreType.DMA,
        ],
        compiler_params=pltpu.CompilerParams(
            needs_layout_passes=False, use_tc_tiling_on_sc=False),
    )
    def k(xa_hbm, src_hbm, dst_hbm, et_hbm, zr_hbm, out_hbm,
          src_strip, dst_strip, et_strip, comp_src, comp_dst,
          src_idx, dst_idx, rows, zbuf, acc, sem):
        cid = lax.axis_index("c")
        tid = lax.axis_index("s")
        e0 = tid * _EPT
        pltpu.sync_copy(zr_hbm, zbuf)
        r0 = tid * _RPT
        for z in range(_RPT // _ZCH):
            pltpu.sync_copy(zbuf, acc.at[pl.ds(r0 + z * _ZCH, _ZCH)])
        plsc.subcore_barrier()

        iota16 = lax.iota(jnp.int32, 16)
        dummy16 = jnp.full((16,), _N, jnp.int32)
        trash16 = jnp.full((16,), _CTRASH, jnp.int32)

        def chunk_body(j, _):
            for v in range(_CH // 16):
                s16 = comp_src[pl.ds(j * _CH + v * 16, 16)]
                d16 = comp_dst[pl.ds(j * _CH + v * 16, 16)]
                src_idx.at[0][pl.ds(v * 16, 16)] = s16
                dst_idx.at[0][pl.ds(v * 16, 16)] = d16
            pltpu.async_copy(xa_hbm.at[src_idx.at[0]], rows, sem).wait()
            pltpu.sync_copy(rows, acc.at[dst_idx.at[0]], add=True)
            return 0

        for p in range(_R // _NC):
            rel = _NC * p + cid
            rel16 = jnp.full((16,), rel, jnp.int32)

            def strip_body(s, kc, rel16=rel16):
                so = e0 + s * _SL
                pltpu.sync_copy(src_hbm.at[pl.ds(so, _SL)], src_strip)
                pltpu.sync_copy(dst_hbm.at[pl.ds(so, _SL)], dst_strip)
                pltpu.sync_copy(et_hbm.at[pl.ds(so, _SL)], et_strip)

                def scan_body(i, kc):
                    et16 = et_strip[pl.ds(i * 16, 16)]
                    m = et16 == rel16
                    mv = m.astype(jnp.int32)
                    excl = plsc.cumsum(mv) - mv
                    tgt = jnp.where(m, jnp.full((16,), kc, jnp.int32) + excl,
                                    trash16)
                    s16 = src_strip[pl.ds(i * 16, 16)]
                    d16 = dst_strip[pl.ds(i * 16, 16)]
                    plsc.store_scatter(comp_src, [tgt], s16)
                    plsc.store_scatter(comp_dst, [tgt], d16)
                    return kc + jnp.sum(mv)

                kc = lax.fori_loop(0, _SL // 16, scan_body, kc)
                nfull = kc // _CH
                lax.fori_loop(0, nfull, chunk_body, 0)
                for v in range(_CH // 16):
                    s16 = comp_src[pl.ds(nfull * _CH + v * 16, 16)]
                    d16 = comp_dst[pl.ds(nfull * _CH + v * 16, 16)]
                    comp_src[pl.ds(v * 16, 16)] = s16
                    comp_dst[pl.ds(v * 16, 16)] = d16
                return kc - nfull * _CH

            kc = lax.fori_loop(0, _NSTRIP, strip_body, jnp.int32(0))
            for v in range(_CH // 16):
                tgt = jnp.full((16,), kc + v * 16, jnp.int32) + iota16
                plsc.store_scatter(comp_src, [tgt], dummy16)
                plsc.store_scatter(comp_dst, [tgt], dummy16)
            nlast = (kc + (_CH - 1)) // _CH
            lax.fori_loop(0, nlast, chunk_body, 0)
            plsc.subcore_barrier()
            for z in range(3):
                sl = pl.ds(r0 + z * _OCH, _OCH)
                pltpu.sync_copy(acc.at[sl], out_hbm.at[rel].at[sl])

            @pl.when(tid < _NS - 1)
            def _():
                for z in range(3, _RPT // _OCH):
                    sl = pl.ds(r0 + z * _OCH, _OCH)
                    pltpu.sync_copy(acc.at[sl], out_hbm.at[rel].at[sl])

            @pl.when(tid == _NS - 1)
            def _():
                sl = pl.ds(_N - 16, 16)
                pltpu.sync_copy(acc.at[sl], out_hbm.at[rel].at[sl])

            for z in range(_RPT // _ZCH):
                sl = pl.ds(r0 + z * _ZCH, _ZCH)
                pltpu.sync_copy(zbuf, acc.at[sl])
            plsc.subcore_barrier()

    return k(xa, src, dst, edge_type, zrows)


def _tc_body(sums_ref, x_ref, wd_ref, root_ref, b_ref, g_ref, bb_ref, out_ref,
             *, residual):
    xb = x_ref[...]
    acc = jnp.dot(xb, root_ref[...], preferred_element_type=jnp.float32)
    acc = acc + b_ref[...]
    for r in range(_R):
        sr = sums_ref[r]
        cnt = jnp.maximum(sr[:, 128:129], 1.0)
        mean = sr[:, :128] / cnt
        acc = acc + jnp.dot(mean, wd_ref[r], preferred_element_type=jnp.float32)
    if residual:
        acc = acc + xb
    mu = jnp.mean(acc, axis=-1, keepdims=True)
    var = jnp.mean((acc - mu) ** 2, axis=-1, keepdims=True)
    y = (acc - mu) * lax.rsqrt(var + 1e-5) * g_ref[...] + bb_ref[...]
    out_ref[...] = jnp.maximum(y, 0.0)


def _tc_layer(sums, x, wd, root, bias, g, bb, *, residual):
    nb = 1000
    grid = (_N // nb,)
    return pl.pallas_call(
        functools.partial(_tc_body, residual=residual),
        grid=grid,
        in_specs=[
            pl.BlockSpec((_R, nb, _W), lambda i: (0, i, 0)),
            pl.BlockSpec((nb, _D), lambda i: (i, 0)),
            pl.BlockSpec((_R, _D, _D), lambda i: (0, 0, 0)),
            pl.BlockSpec((_D, _D), lambda i: (0, 0)),
            pl.BlockSpec((1, _D), lambda i: (0, 0)),
            pl.BlockSpec((1, _D), lambda i: (0, 0)),
            pl.BlockSpec((1, _D), lambda i: (0, 0)),
        ],
        out_specs=pl.BlockSpec((nb, _D), lambda i: (i, 0)),
        out_shape=jax.ShapeDtypeStruct((_N, _D), jnp.float32),
    )(sums, x, wd, root, bias, g, bb)


def _expand_blockdiag(w):
    return jax.vmap(lambda wr: jax.scipy.linalg.block_diag(*[wr[b] for b in range(4)]))(w)


def _augment(x):
    xa = jnp.zeros((_N + 1, _W), jnp.float32)
    xa = xa.at[:_N, :_D].set(x)
    xa = xa.at[:_N, _D].set(1.0)
    return xa


def kernel(edge_index, edge_type, entity_emb, w0, root0, b0, ln_g0, ln_b0,
           w1, root1, b1, ln_g1, ln_b1):
    x = entity_emb
    src = edge_index[0]
    dst = edge_index[1]
    wd0 = _expand_blockdiag(w0)
    wd1 = _expand_blockdiag(w1)
    sums0 = _sc_segment_sums(_augment(x), src, dst, edge_type)
    x1 = _tc_layer(sums0, x, wd0, root0, b0.reshape(1, -1),
                   ln_g0.reshape(1, -1), ln_b0.reshape(1, -1), residual=False)
    sums1 = _sc_segment_sums(_augment(x1), src, dst, edge_type)
    x2 = _tc_layer(sums1, x1, wd1, root1, b1.reshape(1, -1),
                   ln_g1.reshape(1, -1), ln_b1.reshape(1, -1), residual=True)
    return x2


# speedup vs baseline: 2.3037x; 1.0979x over previous
import functools

import jax
import jax.numpy as jnp
from jax import lax
from jax.experimental import pallas as pl
from jax.experimental.pallas import tpu as pltpu
from jax.experimental.pallas import tpu_sc as plsc

_N = 10000
_R = 16
_D = 128
_E = 320000
_W = 144
_NC = 2
_NS = 16
_EPT = _E // _NS
_NA = 10240
_RPT = _NA // _NS
_OCH = 128
_ZCH = 32
_CH = 128
_SL = 4000
_NSTRIP = _EPT // _SL
_CTRASH = 4224
_CCAP = 4240


def _sc_segment_sums(xa, edges_packed):
    mesh = plsc.VectorSubcoreMesh(
        core_axis_name="c", subcore_axis_name="s",
        num_cores=_NC, num_subcores=_NS)
    zrows = jnp.zeros((_ZCH, _W), jnp.float32)

    @functools.partial(
        pl.kernel,
        out_type=jax.ShapeDtypeStruct((_R, _N, _W), jnp.float32),
        mesh=mesh,
        scratch_types=[
            pltpu.VMEM((_SL,), jnp.int32),
            pltpu.VMEM((_CCAP,), jnp.int32),
            pltpu.VMEM((1, _CH), jnp.int32),
            pltpu.VMEM((1, _CH), jnp.int32),
            pltpu.VMEM((_CH, _W), jnp.float32),
            pltpu.VMEM((_ZCH, _W), jnp.float32),
            pltpu.VMEM_SHARED((_NA, _W), jnp.float32),
            pltpu.SemaphoreType.DMA,
            pltpu.SemaphoreType.DMA,
        ],
        compiler_params=pltpu.CompilerParams(
            needs_layout_passes=False, use_tc_tiling_on_sc=False),
    )
    def k(xa_hbm, ep_hbm, zr_hbm, out_hbm,
          strip, comp, src_idx, dst_idx, rows, zbuf, acc, sem, sem2):
        cid = lax.axis_index("c")
        tid = lax.axis_index("s")
        e0 = tid * _EPT
        pltpu.sync_copy(zr_hbm, zbuf)
        r0 = tid * _RPT
        for z in range(_RPT // _ZCH):
            pltpu.async_copy(zbuf, acc.at[pl.ds(r0 + z * _ZCH, _ZCH)], sem2)
        for z in range(_RPT // _ZCH):
            pltpu.make_async_copy(zbuf, acc.at[pl.ds(r0 + z * _ZCH, _ZCH)],
                                  sem2).wait()
        plsc.subcore_barrier()

        iota16 = lax.iota(jnp.int32, 16)
        dummy16 = jnp.full((16,), _N + (_N << 14), jnp.int32)
        trash16 = jnp.full((16,), _CTRASH, jnp.int32)
        m14 = jnp.full((16,), 16383, jnp.int32)
        sh14 = jnp.full((16,), 14, jnp.int32)
        sh28 = jnp.full((16,), 28, jnp.int32)

        def chunk_body(j, _):
            for v in range(_CH // 16):
                pk = comp[pl.ds(j * _CH + v * 16, 16)]
                src_idx.at[0][pl.ds(v * 16, 16)] = pk & m14
                dst_idx.at[0][pl.ds(v * 16, 16)] = (
                    lax.shift_right_logical(pk, sh14) & m14)
            pltpu.async_copy(xa_hbm.at[src_idx.at[0]], rows, sem).wait()
            pltpu.sync_copy(rows, acc.at[dst_idx.at[0]], add=True)
            return 0

        for p in range(_R // _NC):
            rel = _NC * p + cid
            rel16 = jnp.full((16,), rel, jnp.int32)

            def strip_body(s, kc, rel16=rel16):
                so = e0 + s * _SL
                pltpu.sync_copy(ep_hbm.at[pl.ds(so, _SL)], strip)

                def scan_body(i, kc):
                    pk = strip[pl.ds(i * 16, 16)]
                    m = lax.shift_right_logical(pk, sh28) == rel16
                    mv = m.astype(jnp.int32)
                    incl = plsc.cumsum(mv)
                    tgt = jnp.where(
                        m, jnp.full((16,), kc, jnp.int32) + incl - mv, trash16)
                    plsc.store_scatter(comp, [tgt], pk)
                    return kc + incl[15]

                kc = lax.fori_loop(0, _SL // 16, scan_body, kc)
                nfull = kc // _CH
                lax.fori_loop(0, nfull, chunk_body, 0)
                for v in range(_CH // 16):
                    pk = comp[pl.ds(nfull * _CH + v * 16, 16)]
                    comp[pl.ds(v * 16, 16)] = pk
                return kc - nfull * _CH

            kc = lax.fori_loop(0, _NSTRIP, strip_body, jnp.int32(0))
            for v in range(_CH // 16):
                tgt = jnp.full((16,), kc + v * 16, jnp.int32) + iota16
                plsc.store_scatter(comp, [tgt], dummy16)
            nlast = (kc + (_CH - 1)) // _CH
            lax.fori_loop(0, nlast, chunk_body, 0)
            plsc.subcore_barrier()
            for z in range(3):
                sl = pl.ds(r0 + z * _OCH, _OCH)
                pltpu.async_copy(acc.at[sl], out_hbm.at[rel].at[sl], sem2)
            for z in range(3):
                sl = pl.ds(r0 + z * _OCH, _OCH)
                pltpu.make_async_copy(acc.at[sl], out_hbm.at[rel].at[sl],
                                      sem2).wait()

            @pl.when(tid < _NS - 1)
            def _():
                for z in range(3, _RPT // _OCH):
                    sl = pl.ds(r0 + z * _OCH, _OCH)
                    pltpu.async_copy(acc.at[sl], out_hbm.at[rel].at[sl], sem2)
                for z in range(3, _RPT // _OCH):
                    sl = pl.ds(r0 + z * _OCH, _OCH)
                    pltpu.make_async_copy(acc.at[sl], out_hbm.at[rel].at[sl],
                                          sem2).wait()

            @pl.when(tid == _NS - 1)
            def _():
                sl = pl.ds(_N - 16, 16)
                pltpu.sync_copy(acc.at[sl], out_hbm.at[rel].at[sl])

            for z in range(_RPT // _ZCH):
                sl = pl.ds(r0 + z * _ZCH, _ZCH)
                pltpu.async_copy(zbuf, acc.at[sl], sem2)
            for z in range(_RPT // _ZCH):
                sl = pl.ds(r0 + z * _ZCH, _ZCH)
                pltpu.make_async_copy(zbuf, acc.at[sl], sem2).wait()
            plsc.subcore_barrier()

    return k(xa, edges_packed, zrows)


def _tc_body(sums_ref, x_ref, wd_ref, root_ref, b_ref, g_ref, bb_ref, out_ref,
             *, residual):
    xb = x_ref[...]
    acc = jnp.dot(xb, root_ref[...], preferred_element_type=jnp.float32)
    acc = acc + b_ref[...]
    for r in range(_R):
        sr = sums_ref[r]
        cnt = jnp.maximum(sr[:, 128:129], 1.0)
        mean = sr[:, :128] / cnt
        acc = acc + jnp.dot(mean, wd_ref[r], preferred_element_type=jnp.float32)
    if residual:
        acc = acc + xb
    mu = jnp.mean(acc, axis=-1, keepdims=True)
    var = jnp.mean((acc - mu) ** 2, axis=-1, keepdims=True)
    y = (acc - mu) * lax.rsqrt(var + 1e-5) * g_ref[...] + bb_ref[...]
    out_ref[...] = jnp.maximum(y, 0.0)


def _tc_layer(sums, x, wd, root, bias, g, bb, *, residual):
    nb = 1000
    grid = (_N // nb,)
    return pl.pallas_call(
        functools.partial(_tc_body, residual=residual),
        grid=grid,
        in_specs=[
            pl.BlockSpec((_R, nb, _W), lambda i: (0, i, 0)),
            pl.BlockSpec((nb, _D), lambda i: (i, 0)),
            pl.BlockSpec((_R, _D, _D), lambda i: (0, 0, 0)),
            pl.BlockSpec((_D, _D), lambda i: (0, 0)),
            pl.BlockSpec((1, _D), lambda i: (0, 0)),
            pl.BlockSpec((1, _D), lambda i: (0, 0)),
            pl.BlockSpec((1, _D), lambda i: (0, 0)),
        ],
        out_specs=pl.BlockSpec((nb, _D), lambda i: (i, 0)),
        out_shape=jax.ShapeDtypeStruct((_N, _D), jnp.float32),
    )(sums, x, wd, root, bias, g, bb)


def _expand_blockdiag(w):
    return jax.vmap(lambda wr: jax.scipy.linalg.block_diag(*[wr[b] for b in range(4)]))(w)


def _augment(x):
    xa = jnp.zeros((_N + 1, _W), jnp.float32)
    xa = xa.at[:_N, :_D].set(x)
    xa = xa.at[:_N, _D].set(1.0)
    return xa


def kernel(edge_index, edge_type, entity_emb, w0, root0, b0, ln_g0, ln_b0,
           w1, root1, b1, ln_g1, ln_b1):
    x = entity_emb
    edges_packed = ((edge_type << 28) | (edge_index[1] << 14) | edge_index[0])
    wd0 = _expand_blockdiag(w0)
    wd1 = _expand_blockdiag(w1)
    sums0 = _sc_segment_sums(_augment(x), edges_packed)
    x1 = _tc_layer(sums0, x, wd0, root0, b0.reshape(1, -1),
                   ln_g0.reshape(1, -1), ln_b0.reshape(1, -1), residual=False)
    sums1 = _sc_segment_sums(_augment(x1), edges_packed)
    x2 = _tc_layer(sums1, x1, wd1, root1, b1.reshape(1, -1),
                   ln_g1.reshape(1, -1), ln_b1.reshape(1, -1), residual=True)
    return x2


# speedup vs baseline: 2.5998x; 1.1285x over previous
import functools

import jax
import jax.numpy as jnp
from jax import lax
from jax.experimental import pallas as pl
from jax.experimental.pallas import tpu as pltpu
from jax.experimental.pallas import tpu_sc as plsc

_N = 10000
_R = 16
_D = 128
_E = 320000
_W = 144
_NC = 2
_NS = 16
_EPT = _E // _NS
_NA = 10240
_RPT = _NA // _NS
_OCH = 128
_ZCH = 32
_CH = 64
_RING = 2 * _CH
_SL = 4000
_NSTRIP = _EPT // _SL
_CTRASH = _RING
_CCAP = _RING + 16


def _sc_segment_sums(xa, edges_packed):
    mesh = plsc.VectorSubcoreMesh(
        core_axis_name="c", subcore_axis_name="s",
        num_cores=_NC, num_subcores=_NS)
    zrows = jnp.zeros((_ZCH, _W), jnp.float32)

    @functools.partial(
        pl.kernel,
        out_type=jax.ShapeDtypeStruct((_R, _N, _W), jnp.float32),
        mesh=mesh,
        scratch_types=[
            pltpu.VMEM((_SL,), jnp.int32),
            pltpu.VMEM((_CCAP,), jnp.int32),
            pltpu.VMEM((2, _CH), jnp.int32),
            pltpu.VMEM((2, _CH), jnp.int32),
            pltpu.VMEM((2, _CH, _W), jnp.float32),
            pltpu.VMEM((_ZCH, _W), jnp.float32),
            pltpu.VMEM_SHARED((_NA, _W), jnp.float32),
            pltpu.SemaphoreType.DMA,
            pltpu.SemaphoreType.DMA,
            pltpu.SemaphoreType.DMA,
        ],
        compiler_params=pltpu.CompilerParams(
            needs_layout_passes=False, use_tc_tiling_on_sc=False),
    )
    def k(xa_hbm, ep_hbm, zr_hbm, out_hbm,
          strip, comp, src_idx, dst_idx, rows, zbuf, acc, sem_g, sem_s, sem2):
        cid = lax.axis_index("c")
        tid = lax.axis_index("s")
        e0 = tid * _EPT
        pltpu.sync_copy(zr_hbm, zbuf)
        r0 = tid * _RPT
        for z in range(_RPT // _ZCH):
            pltpu.async_copy(zbuf, acc.at[pl.ds(r0 + z * _ZCH, _ZCH)], sem2)
        for z in range(_RPT // _ZCH):
            pltpu.make_async_copy(zbuf, acc.at[pl.ds(r0 + z * _ZCH, _ZCH)],
                                  sem2).wait()
        plsc.subcore_barrier()

        iota16 = lax.iota(jnp.int32, 16)
        dummy16 = jnp.full((16,), _N + (_N << 14), jnp.int32)
        trash16 = jnp.full((16,), _CTRASH, jnp.int32)
        ringm16 = jnp.full((16,), _RING - 1, jnp.int32)
        m14 = jnp.full((16,), 16383, jnp.int32)
        sh14 = jnp.full((16,), 14, jnp.int32)
        sh28 = jnp.full((16,), 28, jnp.int32)

        def wait_gather(q):
            pltpu.make_async_copy(
                xa_hbm.at[src_idx.at[q]], rows.at[q], sem_g).wait()

        def fire_sadd(q):
            pltpu.async_copy(rows.at[q], acc.at[dst_idx.at[q]], sem_s,
                             add=True)

        def wait_sadd(q):
            pltpu.make_async_copy(rows.at[q], acc.at[dst_idx.at[q]],
                                  sem_s).wait()

        def event(c):
            q = c & 1

            @pl.when(c >= 1)
            def _():
                wait_gather(1 - q)
                fire_sadd(1 - q)

            @pl.when(c >= 2)
            def _():
                wait_sadd(q)

            for v in range(_CH // 16):
                pk = comp[pl.ds(q * _CH + v * 16, 16)]
                src_idx.at[q][pl.ds(v * 16, 16)] = pk & m14
                dst_idx.at[q][pl.ds(v * 16, 16)] = (
                    lax.shift_right_logical(pk, sh14) & m14)
            pltpu.async_copy(xa_hbm.at[src_idx.at[q]], rows.at[q], sem_g)

        for p in range(_R // _NC):
            rel = _NC * p + cid
            rel16 = jnp.full((16,), rel, jnp.int32)

            def strip_body(s, kc, rel16=rel16):
                so = e0 + s * _SL
                pltpu.sync_copy(ep_hbm.at[pl.ds(so, _SL)], strip)

                def scan_body(i, kc):
                    pk = strip[pl.ds(i * 16, 16)]
                    m = lax.shift_right_logical(pk, sh28) == rel16
                    mv = m.astype(jnp.int32)
                    incl = plsc.cumsum(mv)
                    tgt = jnp.where(
                        m,
                        (jnp.full((16,), kc, jnp.int32) + incl - mv) & ringm16,
                        trash16)
                    plsc.store_scatter(comp, [tgt], pk)
                    kc_new = kc + incl[15]

                    @pl.when(kc_new // _CH > kc // _CH)
                    def _():
                        event(kc // _CH)

                    return kc_new

                return lax.fori_loop(0, _SL // 16, scan_body, kc)

            kc = lax.fori_loop(0, _NSTRIP, strip_body, jnp.int32(0))
            cs = kc // _CH
            ct = (kc + (_CH - 1)) // _CH
            for v in range(_CH // 16):
                tgt = (jnp.full((16,), kc + v * 16, jnp.int32) + iota16) \
                    & ringm16
                plsc.store_scatter(comp, [tgt], dummy16)

            @pl.when(ct > cs)
            def _():
                event(cs)

            @pl.when(ct >= 1)
            def _():
                wait_gather((ct - 1) & 1)
                fire_sadd((ct - 1) & 1)

            @pl.when(ct >= 2)
            def _():
                wait_sadd((ct - 2) & 1)

            @pl.when(ct >= 1)
            def _():
                wait_sadd((ct - 1) & 1)

            plsc.subcore_barrier()
            for z in range(3):
                sl = pl.ds(r0 + z * _OCH, _OCH)
                pltpu.async_copy(acc.at[sl], out_hbm.at[rel].at[sl], sem2)
            for z in range(3):
                sl = pl.ds(r0 + z * _OCH, _OCH)
                pltpu.make_async_copy(acc.at[sl], out_hbm.at[rel].at[sl],
                                      sem2).wait()

            @pl.when(tid < _NS - 1)
            def _():
                for z in range(3, _RPT // _OCH):
                    sl = pl.ds(r0 + z * _OCH, _OCH)
                    pltpu.async_copy(acc.at[sl], out_hbm.at[rel].at[sl], sem2)
                for z in range(3, _RPT // _OCH):
                    sl = pl.ds(r0 + z * _OCH, _OCH)
                    pltpu.make_async_copy(acc.at[sl], out_hbm.at[rel].at[sl],
                                          sem2).wait()

            @pl.when(tid == _NS - 1)
            def _():
                sl = pl.ds(_N - 16, 16)
                pltpu.sync_copy(acc.at[sl], out_hbm.at[rel].at[sl])

            for z in range(_RPT // _ZCH):
                sl = pl.ds(r0 + z * _ZCH, _ZCH)
                pltpu.async_copy(zbuf, acc.at[sl], sem2)
            for z in range(_RPT // _ZCH):
                sl = pl.ds(r0 + z * _ZCH, _ZCH)
                pltpu.make_async_copy(zbuf, acc.at[sl], sem2).wait()
            plsc.subcore_barrier()

    return k(xa, edges_packed, zrows)


def _tc_body(sums_ref, x_ref, wd_ref, root_ref, b_ref, g_ref, bb_ref, out_ref,
             *, residual):
    xb = x_ref[...]
    acc = jnp.dot(xb, root_ref[...], preferred_element_type=jnp.float32)
    acc = acc + b_ref[...]
    for r in range(_R):
        sr = sums_ref[r]
        cnt = jnp.maximum(sr[:, 128:129], 1.0)
        mean = sr[:, :128] / cnt
        acc = acc + jnp.dot(mean, wd_ref[r], preferred_element_type=jnp.float32)
    if residual:
        acc = acc + xb
    mu = jnp.mean(acc, axis=-1, keepdims=True)
    var = jnp.mean((acc - mu) ** 2, axis=-1, keepdims=True)
    y = (acc - mu) * lax.rsqrt(var + 1e-5) * g_ref[...] + bb_ref[...]
    out_ref[...] = jnp.maximum(y, 0.0)


def _tc_layer(sums, x, wd, root, bias, g, bb, *, residual):
    nb = 1000
    grid = (_N // nb,)
    return pl.pallas_call(
        functools.partial(_tc_body, residual=residual),
        grid=grid,
        in_specs=[
            pl.BlockSpec((_R, nb, _W), lambda i: (0, i, 0)),
            pl.BlockSpec((nb, _D), lambda i: (i, 0)),
            pl.BlockSpec((_R, _D, _D), lambda i: (0, 0, 0)),
            pl.BlockSpec((_D, _D), lambda i: (0, 0)),
            pl.BlockSpec((1, _D), lambda i: (0, 0)),
            pl.BlockSpec((1, _D), lambda i: (0, 0)),
            pl.BlockSpec((1, _D), lambda i: (0, 0)),
        ],
        out_specs=pl.BlockSpec((nb, _D), lambda i: (i, 0)),
        out_shape=jax.ShapeDtypeStruct((_N, _D), jnp.float32),
    )(sums, x, wd, root, bias, g, bb)


def _expand_blockdiag(w):
    return jax.vmap(lambda wr: jax.scipy.linalg.block_diag(*[wr[b] for b in range(4)]))(w)


def _augment(x):
    xa = jnp.zeros((_N + 1, _W), jnp.float32)
    xa = xa.at[:_N, :_D].set(x)
    xa = xa.at[:_N, _D].set(1.0)
    return xa


def kernel(edge_index, edge_type, entity_emb, w0, root0, b0, ln_g0, ln_b0,
           w1, root1, b1, ln_g1, ln_b1):
    x = entity_emb
    edges_packed = ((edge_type << 28) | (edge_index[1] << 14) | edge_index[0])
    wd0 = _expand_blockdiag(w0)
    wd1 = _expand_blockdiag(w1)
    sums0 = _sc_segment_sums(_augment(x), edges_packed)
    x1 = _tc_layer(sums0, x, wd0, root0, b0.reshape(1, -1),
                   ln_g0.reshape(1, -1), ln_b0.reshape(1, -1), residual=False)
    sums1 = _sc_segment_sums(_augment(x1), edges_packed)
    x2 = _tc_layer(sums1, x1, wd1, root1, b1.reshape(1, -1),
                   ln_g1.reshape(1, -1), ln_b1.reshape(1, -1), residual=True)
    return x2


# speedup vs baseline: 3.2211x; 1.2390x over previous
import functools

import jax
import jax.numpy as jnp
from jax import lax
from jax.experimental import pallas as pl
from jax.experimental.pallas import tpu as pltpu
from jax.experimental.pallas import tpu_sc as plsc

_N = 10000
_R = 16
_D = 128
_E = 320000
_W = 144
_NC = 2
_NS = 16
_EPT = _E // _NS
_NA = 10240
_RPT = _NA // _NS
_OCH = 128
_ZCH = 32
_CH = 64
_RING = 2 * _CH
_SL = 4000
_NSTRIP = _EPT // _SL
_CTRASH = _RING
_CCAP = _RING + 16


def _sc_segment_sums(xa, edges_packed):
    mesh = plsc.VectorSubcoreMesh(
        core_axis_name="c", subcore_axis_name="s",
        num_cores=_NC, num_subcores=_NS)
    zrows = jnp.zeros((_ZCH, _W), jnp.float32)

    @functools.partial(
        pl.kernel,
        out_type=jax.ShapeDtypeStruct((_R, _N, _W), jnp.float32),
        mesh=mesh,
        scratch_types=[
            pltpu.VMEM((2, _SL), jnp.int32),
            pltpu.VMEM((_CCAP,), jnp.int32),
            pltpu.VMEM((2, _CH), jnp.int32),
            pltpu.VMEM((2, _CH), jnp.int32),
            pltpu.VMEM((2, _CH, _W), jnp.float32),
            pltpu.VMEM((_ZCH, _W), jnp.float32),
            pltpu.VMEM_SHARED((_NA, _W), jnp.float32),
            pltpu.SemaphoreType.DMA,
            pltpu.SemaphoreType.DMA,
            pltpu.SemaphoreType.DMA,
            pltpu.SemaphoreType.DMA,
        ],
        compiler_params=pltpu.CompilerParams(
            needs_layout_passes=False, use_tc_tiling_on_sc=False),
    )
    def k(xa_hbm, ep_hbm, zr_hbm, out_hbm,
          strip, comp, src_idx, dst_idx, rows, zbuf, acc,
          sem_g, sem_s, sem2, sem_l):
        cid = lax.axis_index("c")
        tid = lax.axis_index("s")
        e0 = tid * _EPT
        pltpu.sync_copy(zr_hbm, zbuf)
        r0 = tid * _RPT
        for z in range(_RPT // _ZCH):
            pltpu.async_copy(zbuf, acc.at[pl.ds(r0 + z * _ZCH, _ZCH)], sem2)
        for z in range(_RPT // _ZCH):
            pltpu.make_async_copy(zbuf, acc.at[pl.ds(r0 + z * _ZCH, _ZCH)],
                                  sem2).wait()
        plsc.subcore_barrier()

        iota16 = lax.iota(jnp.int32, 16)
        dummy16 = jnp.full((16,), _N + (_N << 14), jnp.int32)
        trash16 = jnp.full((16,), _CTRASH, jnp.int32)
        ringm16 = jnp.full((16,), _RING - 1, jnp.int32)
        m14 = jnp.full((16,), 16383, jnp.int32)
        sh14 = jnp.full((16,), 14, jnp.int32)
        sh28 = jnp.full((16,), 28, jnp.int32)

        def wait_gather(q):
            pltpu.make_async_copy(
                xa_hbm.at[src_idx.at[q]], rows.at[q], sem_g).wait()

        def fire_sadd(q):
            pltpu.async_copy(rows.at[q], acc.at[dst_idx.at[q]], sem_s,
                             add=True)

        def wait_sadd(q):
            pltpu.make_async_copy(rows.at[q], acc.at[dst_idx.at[q]],
                                  sem_s).wait()

        def event(c):
            q = c & 1

            @pl.when(c >= 1)
            def _():
                wait_gather(1 - q)
                fire_sadd(1 - q)

            @pl.when(c >= 2)
            def _():
                wait_sadd(q)

            for v in range(_CH // 16):
                pk = comp[pl.ds(q * _CH + v * 16, 16)]
                src_idx.at[q][pl.ds(v * 16, 16)] = pk & m14
                dst_idx.at[q][pl.ds(v * 16, 16)] = (
                    lax.shift_right_logical(pk, sh14) & m14)
            pltpu.async_copy(xa_hbm.at[src_idx.at[q]], rows.at[q], sem_g)

        def fire_strip_load(s):
            pltpu.async_copy(ep_hbm.at[pl.ds(e0 + s * _SL, _SL)],
                             strip.at[s % 2], sem_l)

        def wait_strip_load(s):
            pltpu.make_async_copy(ep_hbm.at[pl.ds(e0 + s * _SL, _SL)],
                                  strip.at[s % 2], sem_l).wait()

        for p in range(_R // _NC):
            rel = _NC * p + cid
            rel16 = jnp.full((16,), rel, jnp.int32)

            fire_strip_load(0)

            def strip_body(s, kc, rel16=rel16):
                sb = strip.at[s % 2]
                wait_strip_load(s)

                @pl.when(s + 1 < _NSTRIP)
                def _():
                    fire_strip_load(s + 1)

                def scan_half(pk, kc, rel16=rel16):
                    m = lax.shift_right_logical(pk, sh28) == rel16
                    mv = m.astype(jnp.int32)
                    incl = plsc.cumsum(mv)
                    tgt = jnp.where(
                        m,
                        (jnp.full((16,), kc, jnp.int32) + incl - mv) & ringm16,
                        trash16)
                    plsc.store_scatter(comp, [tgt], pk)
                    return kc + incl[15]

                def scan_body(i, kc, sb=sb):
                    pk_a = sb[pl.ds(i * 32, 16)]
                    pk_b = sb[pl.ds(i * 32 + 16, 16)]
                    kc1 = scan_half(pk_a, kc)
                    kc2 = scan_half(pk_b, kc1)

                    @pl.when(kc2 // _CH > kc // _CH)
                    def _():
                        event(kc // _CH)

                    return kc2

                return lax.fori_loop(0, _SL // 32, scan_body, kc)

            kc = lax.fori_loop(0, _NSTRIP, strip_body, jnp.int32(0))
            cs = kc // _CH
            ct = (kc + (_CH - 1)) // _CH
            for v in range(_CH // 16):
                tgt = (jnp.full((16,), kc + v * 16, jnp.int32) + iota16) \
                    & ringm16
                plsc.store_scatter(comp, [tgt], dummy16)

            @pl.when(ct > cs)
            def _():
                event(cs)

            @pl.when(ct >= 1)
            def _():
                wait_gather((ct - 1) & 1)
                fire_sadd((ct - 1) & 1)

            @pl.when(ct >= 2)
            def _():
                wait_sadd((ct - 2) & 1)

            @pl.when(ct >= 1)
            def _():
                wait_sadd((ct - 1) & 1)

            plsc.subcore_barrier()
            for z in range(3):
                sl = pl.ds(r0 + z * _OCH, _OCH)
                pltpu.async_copy(acc.at[sl], out_hbm.at[rel].at[sl], sem2)
            for z in range(3):
                sl = pl.ds(r0 + z * _OCH, _OCH)
                pltpu.make_async_copy(acc.at[sl], out_hbm.at[rel].at[sl],
                                      sem2).wait()

            @pl.when(tid < _NS - 1)
            def _():
                for z in range(3, _RPT // _OCH):
                    sl = pl.ds(r0 + z * _OCH, _OCH)
                    pltpu.async_copy(acc.at[sl], out_hbm.at[rel].at[sl], sem2)
                for z in range(3, _RPT // _OCH):
                    sl = pl.ds(r0 + z * _OCH, _OCH)
                    pltpu.make_async_copy(acc.at[sl], out_hbm.at[rel].at[sl],
                                          sem2).wait()

            @pl.when(tid == _NS - 1)
            def _():
                sl = pl.ds(_N - 16, 16)
                pltpu.sync_copy(acc.at[sl], out_hbm.at[rel].at[sl])

            for z in range(_RPT // _ZCH):
                sl = pl.ds(r0 + z * _ZCH, _ZCH)
                pltpu.async_copy(zbuf, acc.at[sl], sem2)
            for z in range(_RPT // _ZCH):
                sl = pl.ds(r0 + z * _ZCH, _ZCH)
                pltpu.make_async_copy(zbuf, acc.at[sl], sem2).wait()
            plsc.subcore_barrier()

    return k(xa, edges_packed, zrows)


def _tc_body(sums_ref, x_ref, wd_ref, root_ref, b_ref, g_ref, bb_ref, out_ref,
             *, residual):
    xb = x_ref[...]
    acc = jnp.dot(xb, root_ref[...], preferred_element_type=jnp.float32)
    acc = acc + b_ref[...]
    for r in range(_R):
        sr = sums_ref[r]
        cnt = jnp.maximum(sr[:, 128:129], 1.0)
        mean = sr[:, :128] / cnt
        acc = acc + jnp.dot(mean, wd_ref[r], preferred_element_type=jnp.float32)
    if residual:
        acc = acc + xb
    mu = jnp.mean(acc, axis=-1, keepdims=True)
    var = jnp.mean((acc - mu) ** 2, axis=-1, keepdims=True)
    y = (acc - mu) * lax.rsqrt(var + 1e-5) * g_ref[...] + bb_ref[...]
    out_ref[...] = jnp.maximum(y, 0.0)


def _tc_layer(sums, x, wd, root, bias, g, bb, *, residual):
    nb = 1000
    grid = (_N // nb,)
    return pl.pallas_call(
        functools.partial(_tc_body, residual=residual),
        grid=grid,
        in_specs=[
            pl.BlockSpec((_R, nb, _W), lambda i: (0, i, 0)),
            pl.BlockSpec((nb, _D), lambda i: (i, 0)),
            pl.BlockSpec((_R, _D, _D), lambda i: (0, 0, 0)),
            pl.BlockSpec((_D, _D), lambda i: (0, 0)),
            pl.BlockSpec((1, _D), lambda i: (0, 0)),
            pl.BlockSpec((1, _D), lambda i: (0, 0)),
            pl.BlockSpec((1, _D), lambda i: (0, 0)),
        ],
        out_specs=pl.BlockSpec((nb, _D), lambda i: (i, 0)),
        out_shape=jax.ShapeDtypeStruct((_N, _D), jnp.float32),
    )(sums, x, wd, root, bias, g, bb)


def _expand_blockdiag(w):
    return jax.vmap(lambda wr: jax.scipy.linalg.block_diag(*[wr[b] for b in range(4)]))(w)


def _augment(x):
    xa = jnp.zeros((_N + 1, _W), jnp.float32)
    xa = xa.at[:_N, :_D].set(x)
    xa = xa.at[:_N, _D].set(1.0)
    return xa


def kernel(edge_index, edge_type, entity_emb, w0, root0, b0, ln_g0, ln_b0,
           w1, root1, b1, ln_g1, ln_b1):
    x = entity_emb
    edges_packed = ((edge_type << 28) | (edge_index[1] << 14) | edge_index[0])
    wd0 = _expand_blockdiag(w0)
    wd1 = _expand_blockdiag(w1)
    sums0 = _sc_segment_sums(_augment(x), edges_packed)
    x1 = _tc_layer(sums0, x, wd0, root0, b0.reshape(1, -1),
                   ln_g0.reshape(1, -1), ln_b0.reshape(1, -1), residual=False)
    sums1 = _sc_segment_sums(_augment(x1), edges_packed)
    x2 = _tc_layer(sums1, x1, wd1, root1, b1.reshape(1, -1),
                   ln_g1.reshape(1, -1), ln_b1.reshape(1, -1), residual=True)
    return x2
